# baseline (speedup 1.0000x reference)
import dataclasses
import functools

import jax
import jax.numpy as jnp
from jax import lax
from jax.experimental import pallas as pl
from jax.experimental.pallas import tpu as pltpu
from jax.experimental.pallas import tpu_sc as plsc

N = 50000
E = 1600000
D = 512
NCLS = 10
NPASS = 4

LANES = 16
HPAD = 51200
NW = 32
EPW = E // NW
CH = 2000
NCH = EPW // CH

_mesh = plsc.VectorSubcoreMesh(core_axis_name="c", subcore_axis_name="s")

_sc_params = pltpu.CompilerParams()
for _f, _v in (("needs_layout_passes", False), ("use_tc_tiling_on_sc", False)):
    if _f in pltpu.CompilerParams.__dataclass_fields__:
        _sc_params = dataclasses.replace(_sc_params, **{_f: _v})


@functools.partial(
    pl.kernel,
    compiler_params=_sc_params,
    out_type=(
        jax.ShapeDtypeStruct((NW * HPAD,), jnp.float32),
        jax.ShapeDtypeStruct((NW, D), jnp.float32),
    ),
    mesh=_mesh,
    scratch_types=[
        pltpu.VMEM((HPAD,), jnp.float32),
        pltpu.VMEM((HPAD,), jnp.float32),
        pltpu.VMEM((CH,), jnp.int32),
        pltpu.VMEM((CH,), jnp.float32),
        pltpu.VMEM((CH,), jnp.int32),
        pltpu.VMEM((CH,), jnp.float32),
        pltpu.VMEM((CH,), jnp.int32),
        pltpu.VMEM((CH,), jnp.float32),
        pltpu.VMEM((CH,), jnp.int32),
        pltpu.VMEM((CH,), jnp.float32),
        pltpu.VMEM((D,), jnp.int32),
        pltpu.VMEM((D,), jnp.float32),
        pltpu.SemaphoreType.DMA,
        pltpu.SemaphoreType.DMA,
        pltpu.SemaphoreType.DMA,
        pltpu.SemaphoreType.DMA,
        pltpu.SemaphoreType.DMA,
    ],
)
def _sc_pass(h_hbm, ids_hbm, ew_hbm, dm_hbm, part_hbm, selp_hbm,
             h_old, h_acc, ib0, wb0, ib1, wb1, ib2, wb2, ib3, wb3,
             dmbuf, selbuf, sem_h, se0, se1, se2, se3):
    cid = lax.axis_index("c")
    sid = lax.axis_index("s")
    wid = sid * 2 + cid
    ebase = wid * EPW

    def start(c, ib, wb, sem):
        off = ebase + c * CH
        pltpu.async_copy(ids_hbm.at[pl.ds(off, CH)], ib, sem)
        pltpu.async_copy(ew_hbm.at[pl.ds(off, CH)], wb, sem)

    def wait(ib, wb, sem):
        pltpu.make_async_copy(ids_hbm.at[pl.ds(0, CH)], ib, sem).wait()
        pltpu.make_async_copy(ew_hbm.at[pl.ds(0, CH)], wb, sem).wait()

    def one_group(ib, wb, o):
        p16 = ib[pl.ds(o, LANES)]
        w16 = wb[pl.ds(o, LANES)]
        s16 = p16 & 0xFFFF
        d16 = lax.shift_right_logical(p16, 16)
        g = plsc.load_gather(h_old, [s16])
        plsc.addupdate_scatter(h_acc, [d16], g * w16)

    def compute(ib, wb):
        @plsc.parallel_loop(0, CH, LANES, unroll=5)
        def _edges(i):
            one_group(ib, wb, i)

    cp_h = pltpu.async_copy(h_hbm, h_old, sem_h)
    cp_dm = pltpu.async_copy(dm_hbm, dmbuf, sem_h)
    start(0, ib0, wb0, se0)
    start(1, ib1, wb1, se1)
    start(2, ib2, wb2, se2)
    start(3, ib3, wb3, se3)

    @plsc.parallel_loop(0, HPAD, LANES, unroll=8)
    def _zero(j):
        h_acc[pl.ds(j, LANES)] = jnp.zeros((LANES,), jnp.float32)

    cp_h.wait()
    cp_dm.wait()

    @pl.loop(0, NCH - 5, step=4)
    def _chunk(c):
        for k, (ib, wb, sem) in enumerate(
                ((ib0, wb0, se0), (ib1, wb1, se1),
                 (ib2, wb2, se2), (ib3, wb3, se3))):
            wait(ib, wb, sem)
            compute(ib, wb)
            start(c + 4 + k, ib, wb, sem)

    wait(ib0, wb0, se0)
    compute(ib0, wb0)
    start(NCH - 1, ib0, wb0, se0)
    for ib, wb, sem in ((ib1, wb1, se1), (ib2, wb2, se2),
                        (ib3, wb3, se3), (ib0, wb0, se0)):
        wait(ib, wb, sem)
        compute(ib, wb)

    @plsc.parallel_loop(0, D, LANES, unroll=4)
    def _sel(k):
        i16 = dmbuf[pl.ds(k, LANES)]
        selbuf[pl.ds(k, LANES)] = plsc.load_gather(h_acc, [i16])

    pltpu.sync_copy(h_acc, part_hbm.at[pl.ds(wid * HPAD, HPAD)])
    pltpu.sync_copy(selbuf, selp_hbm.at[wid])


def _tc_pack_ids(eidx, x):
    def body(i_ref, x_ref, ids_ref, h_ref):
        ids_ref[...] = i_ref[0, :] | (i_ref[1, :] << 16)
        h_ref[pl.ds(0, N)] = x_ref[:, 0]
        h_ref[pl.ds(N, HPAD - N)] = jnp.zeros((HPAD - N,), jnp.float32)

    return pl.pallas_call(
        body,
        out_shape=(jax.ShapeDtypeStruct((E,), jnp.int32),
                   jax.ShapeDtypeStruct((HPAD,), jnp.float32)),
    )(eidx, x)


def _tc_effw(w, m):
    def body(w_ref, m_ref, ew_ref):
        ew_ref[...] = w_ref[...] * jax.nn.sigmoid(m_ref[...])

    return pl.pallas_call(
        body,
        out_shape=jax.ShapeDtypeStruct((E,), jnp.float32),
    )(w, m)


def _tc_reduce(parts):
    def body(p_ref, o_ref):
        acc = p_ref[pl.ds(0, HPAD)]
        for w in range(1, NW):
            acc = acc + p_ref[pl.ds(w * HPAD, HPAD)]
        o_ref[...] = acc

    return pl.pallas_call(
        body,
        out_shape=jax.ShapeDtypeStruct((HPAD,), jnp.float32),
    )(parts)


def _tc_head(selp, fc_W, fc_b):
    def body(sp_ref, w_ref, b_ref, o_ref):
        s = jnp.sum(sp_ref[...], axis=0, keepdims=True)
        nrm = jnp.sqrt(jnp.sum(s * s))
        y = lax.dot_general(s, w_ref[...], (((1,), (1,)), ((), ())),
                            precision=lax.Precision.HIGHEST,
                            preferred_element_type=jnp.float32)
        o_ref[...] = y / nrm + b_ref[...]

    out = pl.pallas_call(
        body,
        out_shape=jax.ShapeDtypeStruct((1, NCLS), jnp.float32),
    )(selp, fc_W, fc_b.reshape(1, NCLS))
    return out.reshape(NCLS)


def kernel(x, edge_index, edge_weight, edge_weight_multiplier, dm_indices,
           fc_W, fc_b):
    ids, h = _tc_pack_ids(edge_index, x)
    effw = _tc_effw(edge_weight, edge_weight_multiplier)

    selp = None
    for p in range(NPASS):
        parts, selp = _sc_pass(h, ids, effw, dm_indices)
        if p < NPASS - 1:
            h = _tc_reduce(parts)

    return _tc_head(selp, fc_W, fc_b)


# speedup vs baseline: 1.3085x; 1.3085x over previous
import dataclasses
import functools

import jax
import jax.numpy as jnp
from jax import lax
from jax.experimental import pallas as pl
from jax.experimental.pallas import tpu as pltpu
from jax.experimental.pallas import tpu_sc as plsc

N = 50000
E = 1600000
D = 512
NCLS = 10
NPASS = 4

LANES = 16
HPAD = 51200
NW = 32
EPW = E // NW
CH = 2000
NCH = EPW // CH

_mesh = plsc.VectorSubcoreMesh(core_axis_name="c", subcore_axis_name="s")

_sc_params = pltpu.CompilerParams()
for _f, _v in (("needs_layout_passes", False), ("use_tc_tiling_on_sc", False)):
    if _f in pltpu.CompilerParams.__dataclass_fields__:
        _sc_params = dataclasses.replace(_sc_params, **{_f: _v})


@functools.partial(
    pl.kernel,
    compiler_params=_sc_params,
    out_type=(
        jax.ShapeDtypeStruct((NW * HPAD,), jnp.float32),
        jax.ShapeDtypeStruct((NW, D), jnp.float32),
    ),
    mesh=_mesh,
    scratch_types=[
        pltpu.VMEM((HPAD,), jnp.float32),
        pltpu.VMEM((HPAD,), jnp.float32),
        pltpu.VMEM((CH,), jnp.int32),
        pltpu.VMEM((CH,), jnp.float32),
        pltpu.VMEM((CH,), jnp.int32),
        pltpu.VMEM((CH,), jnp.float32),
        pltpu.VMEM((CH,), jnp.int32),
        pltpu.VMEM((CH,), jnp.float32),
        pltpu.VMEM((CH,), jnp.int32),
        pltpu.VMEM((CH,), jnp.float32),
        pltpu.VMEM((D,), jnp.int32),
        pltpu.VMEM((D,), jnp.float32),
        pltpu.SemaphoreType.DMA,
        pltpu.SemaphoreType.DMA,
        pltpu.SemaphoreType.DMA,
        pltpu.SemaphoreType.DMA,
        pltpu.SemaphoreType.DMA,
    ],
)
def _sc_pass(h_hbm, ids_hbm, ew_hbm, dm_hbm, part_hbm, selp_hbm,
             h_old, h_acc, ib0, wb0, ib1, wb1, ib2, wb2, ib3, wb3,
             dmbuf, selbuf, sem_h, se0, se1, se2, se3):
    cid = lax.axis_index("c")
    sid = lax.axis_index("s")
    wid = sid * 2 + cid
    ebase = wid * EPW

    def start(c, ib, wb, sem):
        off = ebase + c * CH
        pltpu.async_copy(ids_hbm.at[pl.ds(off, CH)], ib, sem)
        pltpu.async_copy(ew_hbm.at[pl.ds(off, CH)], wb, sem)

    def wait(ib, wb, sem):
        pltpu.make_async_copy(ids_hbm.at[pl.ds(0, CH)], ib, sem).wait()
        pltpu.make_async_copy(ew_hbm.at[pl.ds(0, CH)], wb, sem).wait()

    def one_group(ib, wb, o):
        p16 = ib[pl.ds(o, LANES)]
        w16 = wb[pl.ds(o, LANES)]
        s16 = p16 & 0xFFFF
        d16 = lax.shift_right_logical(p16, 16)
        g = plsc.load_gather(h_old, [s16])
        plsc.addupdate_scatter(h_acc, [d16], g * w16)

    def compute(ib, wb):
        @plsc.parallel_loop(0, CH, LANES, unroll=5)
        def _edges(i):
            one_group(ib, wb, i)

    cp_h = pltpu.async_copy(h_hbm, h_old, sem_h)
    cp_dm = pltpu.async_copy(dm_hbm, dmbuf, sem_h)
    start(0, ib0, wb0, se0)
    start(1, ib1, wb1, se1)
    start(2, ib2, wb2, se2)
    start(3, ib3, wb3, se3)

    @plsc.parallel_loop(0, HPAD, LANES, unroll=8)
    def _zero(j):
        h_acc[pl.ds(j, LANES)] = jnp.zeros((LANES,), jnp.float32)

    cp_h.wait()
    cp_dm.wait()

    @pl.loop(0, NCH - 5, step=4)
    def _chunk(c):
        for k, (ib, wb, sem) in enumerate(
                ((ib0, wb0, se0), (ib1, wb1, se1),
                 (ib2, wb2, se2), (ib3, wb3, se3))):
            wait(ib, wb, sem)
            compute(ib, wb)
            start(c + 4 + k, ib, wb, sem)

    wait(ib0, wb0, se0)
    compute(ib0, wb0)
    start(NCH - 1, ib0, wb0, se0)
    for ib, wb, sem in ((ib1, wb1, se1), (ib2, wb2, se2),
                        (ib3, wb3, se3), (ib0, wb0, se0)):
        wait(ib, wb, sem)
        compute(ib, wb)

    @plsc.parallel_loop(0, D, LANES, unroll=4)
    def _sel(k):
        i16 = dmbuf[pl.ds(k, LANES)]
        selbuf[pl.ds(k, LANES)] = plsc.load_gather(h_acc, [i16])

    pltpu.sync_copy(h_acc, part_hbm.at[pl.ds(wid * HPAD, HPAD)])
    pltpu.sync_copy(selbuf, selp_hbm.at[wid])


def _tc_pack(eidx, w, m):
    def body(i_ref, w_ref, m_ref, ids_ref, ew_ref):
        ids_ref[...] = i_ref[0, :] | (i_ref[1, :] << 16)
        ew_ref[...] = w_ref[...] * jax.nn.sigmoid(m_ref[...])

    return pl.pallas_call(
        body,
        out_shape=(jax.ShapeDtypeStruct((E,), jnp.int32),
                   jax.ShapeDtypeStruct((E,), jnp.float32)),
    )(eidx, w, m)


def _tc_reduce(parts):
    def body(p_ref, o_ref):
        acc = p_ref[pl.ds(0, HPAD)]
        for w in range(1, NW):
            acc = acc + p_ref[pl.ds(w * HPAD, HPAD)]
        o_ref[...] = acc

    return pl.pallas_call(
        body,
        out_shape=jax.ShapeDtypeStruct((HPAD,), jnp.float32),
    )(parts)


def _tc_head(selp, fc_W, fc_b):
    def body(sp_ref, w_ref, b_ref, o_ref):
        s = jnp.sum(sp_ref[...], axis=0, keepdims=True)
        nrm = jnp.sqrt(jnp.sum(s * s))
        y = lax.dot_general(s, w_ref[...], (((1,), (1,)), ((), ())),
                            precision=lax.Precision.HIGHEST,
                            preferred_element_type=jnp.float32)
        o_ref[...] = y / nrm + b_ref[...]

    out = pl.pallas_call(
        body,
        out_shape=jax.ShapeDtypeStruct((1, NCLS), jnp.float32),
    )(selp, fc_W, fc_b.reshape(1, NCLS))
    return out.reshape(NCLS)


def kernel(x, edge_index, edge_weight, edge_weight_multiplier, dm_indices,
           fc_W, fc_b):
    ids, effw = _tc_pack(edge_index, edge_weight, edge_weight_multiplier)

    h = jnp.zeros((HPAD,), jnp.float32).at[:N].set(x.reshape(-1))

    selp = None
    for p in range(NPASS):
        parts, selp = _sc_pass(h, ids, effw, dm_indices)
        if p < NPASS - 1:
            h = _tc_reduce(parts)

    return _tc_head(selp, fc_W, fc_b)
